# initial kernel scaffold (unmeasured)
import jax
import jax.numpy as jnp
from jax import lax
from jax.experimental import pallas as pl
from jax.experimental.pallas import tpu as pltpu

N_DEV = 4


def kernel(x, w_mat, scale_x, scale_w):
    m, k_per = x.shape
    _, n = w_mat.shape
    m_per = m // N_DEV

    def body(x_ref, w_ref, sx_ref, sw_ref, out_ref, rs_buf,
             send_sems, recv_sems):
        my = lax.axis_index("i")
        left = (my - 1) % N_DEV
        right = (my + 1) % N_DEV

        barrier_sem = pltpu.get_barrier_semaphore()
        for nbr in (left, right):
            pl.semaphore_signal(
                barrier_sem, inc=1,
                device_id=(nbr,), device_id_type=pl.DeviceIdType.MESH,
            )
        pl.semaphore_wait(barrier_sem, 2)

        out_ref[...] = jnp.dot(x_ref[...], w_ref[...],
                               preferred_element_type=jnp.float32)

        for s in range(N_DEV - 1):
            send_c = (my - s) % N_DEV
            recv_c = (my - 1 - s) % N_DEV
            rdma = pltpu.make_async_remote_copy(
                src_ref=out_ref.at[pl.ds(send_c * m_per, m_per), :],
                dst_ref=rs_buf.at[s],
                send_sem=send_sems.at[s],
                recv_sem=recv_sems.at[s],
                device_id=(right,),
                device_id_type=pl.DeviceIdType.MESH,
            )
            rdma.start()
            rdma.wait()
            out_ref[pl.ds(recv_c * m_per, m_per), :] += rs_buf[s]

        scale = sx_ref[0] * sw_ref[0]
        own_c = (my + 1) % N_DEV
        own = out_ref[pl.ds(own_c * m_per, m_per), :]
        out_ref[pl.ds(own_c * m_per, m_per), :] = jnp.maximum(own * scale, 0.0)

        for t in range(N_DEV - 1):
            send_c = (my + 1 - t) % N_DEV
            rdma = pltpu.make_async_remote_copy(
                src_ref=out_ref.at[pl.ds(send_c * m_per, m_per), :],
                dst_ref=out_ref.at[pl.ds(send_c * m_per, m_per), :],
                send_sem=send_sems.at[N_DEV - 1 + t],
                recv_sem=recv_sems.at[N_DEV - 1 + t],
                device_id=(right,),
                device_id_type=pl.DeviceIdType.MESH,
            )
            rdma.start()
            rdma.wait()

    return pl.pallas_call(
        body,
        out_shape=jax.ShapeDtypeStruct((m, n), jnp.float32),
        in_specs=[
            pl.BlockSpec(memory_space=pltpu.VMEM),
            pl.BlockSpec(memory_space=pltpu.VMEM),
            pl.BlockSpec(memory_space=pltpu.SMEM),
            pl.BlockSpec(memory_space=pltpu.SMEM),
        ],
        out_specs=pl.BlockSpec(memory_space=pltpu.VMEM),
        scratch_shapes=[
            pltpu.VMEM((N_DEV - 1, m_per, n), jnp.float32),
            pltpu.SemaphoreType.DMA((2 * (N_DEV - 1),)),
            pltpu.SemaphoreType.DMA((2 * (N_DEV - 1),)),
        ],
        compiler_params=pltpu.CompilerParams(collective_id=0),
    )(x, w_mat, scale_x, scale_w)


# baseline (device time: 353426 ns/iter reference)
import jax
import jax.numpy as jnp
from jax import lax
from jax.experimental import pallas as pl
from jax.experimental.pallas import tpu as pltpu

N_DEV = 4


def kernel(x, w_mat, scale_x, scale_w):
    m, k_per = x.shape
    _, n = w_mat.shape
    m_per = m // N_DEV

    def body(x_hbm, w_ref, sx_ref, sw_ref, out_hbm,
             xchunk, pbuf, rs_buf, stage, load_sem, store_sem,
             send_sems, recv_sems):
        my = lax.axis_index("i")
        left = (my - 1) % N_DEV
        right = (my + 1) % N_DEV

        barrier_sem = pltpu.get_barrier_semaphore()
        for nbr in (left, right):
            pl.semaphore_signal(
                barrier_sem, inc=1,
                device_id=(nbr,), device_id_type=pl.DeviceIdType.MESH,
            )
        pl.semaphore_wait(barrier_sem, 2)

        for c in range(N_DEV):
            cp = pltpu.make_async_copy(
                x_hbm.at[pl.ds(c * m_per, m_per), :], xchunk, load_sem)
            cp.start()
            cp.wait()
            pbuf[pl.ds(c * m_per, m_per), :] = jnp.dot(
                xchunk[...], w_ref[...],
                preferred_element_type=jnp.float32).astype(jnp.bfloat16)

        for s in range(N_DEV - 1):
            send_c = (my - s) % N_DEV
            recv_c = (my - 1 - s) % N_DEV
            rdma = pltpu.make_async_remote_copy(
                src_ref=pbuf.at[pl.ds(send_c * m_per, m_per), :],
                dst_ref=rs_buf.at[s],
                send_sem=send_sems.at[s],
                recv_sem=recv_sems.at[s],
                device_id=(right,),
                device_id_type=pl.DeviceIdType.MESH,
            )
            rdma.start()
            rdma.wait()
            pbuf[pl.ds(recv_c * m_per, m_per), :] += rs_buf[s]

        scale = (sx_ref[0] * sw_ref[0]).astype(jnp.bfloat16)
        own_c = (my + 1) % N_DEV
        own = pbuf[pl.ds(own_c * m_per, m_per), :]
        pbuf[pl.ds(own_c * m_per, m_per), :] = jnp.maximum(own * scale, 0)

        def store_chunk(c):
            stage[...] = pbuf[pl.ds(c * m_per, m_per), :].astype(jnp.float32)
            cp = pltpu.make_async_copy(
                stage, out_hbm.at[pl.ds(c * m_per, m_per), :], store_sem)
            cp.start()
            cp.wait()

        store_chunk(own_c)

        for t in range(N_DEV - 1):
            send_c = (my + 1 - t) % N_DEV
            recv_c = (my - t) % N_DEV
            rdma = pltpu.make_async_remote_copy(
                src_ref=pbuf.at[pl.ds(send_c * m_per, m_per), :],
                dst_ref=pbuf.at[pl.ds(send_c * m_per, m_per), :],
                send_sem=send_sems.at[N_DEV - 1 + t],
                recv_sem=recv_sems.at[N_DEV - 1 + t],
                device_id=(right,),
                device_id_type=pl.DeviceIdType.MESH,
            )
            rdma.start()
            rdma.wait()
            store_chunk(recv_c)

    return pl.pallas_call(
        body,
        out_shape=jax.ShapeDtypeStruct((m, n), jnp.float32),
        in_specs=[
            pl.BlockSpec(memory_space=pl.ANY),
            pl.BlockSpec(memory_space=pltpu.VMEM),
            pl.BlockSpec(memory_space=pltpu.SMEM),
            pl.BlockSpec(memory_space=pltpu.SMEM),
        ],
        out_specs=pl.BlockSpec(memory_space=pl.ANY),
        scratch_shapes=[
            pltpu.VMEM((m_per, k_per), jnp.float32),
            pltpu.VMEM((m, n), jnp.bfloat16),
            pltpu.VMEM((N_DEV - 1, m_per, n), jnp.bfloat16),
            pltpu.VMEM((m_per, n), jnp.float32),
            pltpu.SemaphoreType.DMA,
            pltpu.SemaphoreType.DMA,
            pltpu.SemaphoreType.DMA((2 * (N_DEV - 1),)),
            pltpu.SemaphoreType.DMA((2 * (N_DEV - 1),)),
        ],
        compiler_params=pltpu.CompilerParams(
            collective_id=0, vmem_limit_bytes=64 * 1024 * 1024),
    )(x, w_mat, scale_x, scale_w)


# device time: 186689 ns/iter; 1.8931x vs baseline; 1.8931x over previous
import jax
import jax.numpy as jnp
from jax import lax
from jax.experimental import pallas as pl
from jax.experimental.pallas import tpu as pltpu

N_DEV = 4
N_RS = N_DEV - 1


def kernel(x, w_mat, scale_x, scale_w):
    m, k_per = x.shape
    _, n = w_mat.shape
    m_per = m // N_DEV
    n_half = n // 2

    def body(x_hbm, w_ref, sx_ref, sw_ref, out_hbm,
             xchunk, pbuf, rs_buf, stage, load_sems, store_sems,
             send_sems, recv_sems):
        my = lax.axis_index("i")
        left = (my - 1) % N_DEV
        right = (my + 1) % N_DEV

        def rows(c):
            return pl.ds(c * m_per, m_per)

        def cols(r):
            return pl.ds(r * n_half, n_half)

        barrier_sem = pltpu.get_barrier_semaphore()
        for nbr in (left, right):
            pl.semaphore_signal(
                barrier_sem, inc=1,
                device_id=(nbr,), device_id_type=pl.DeviceIdType.MESH,
            )
        pl.semaphore_wait(barrier_sem, 2)

        def load(c, slot):
            cp = pltpu.make_async_copy(
                x_hbm.at[rows(c), :], xchunk.at[slot], load_sems.at[slot])
            cp.start()
            return cp

        def compute(c, slot):
            pbuf[rows(c), :] = jnp.dot(
                xchunk[slot], w_ref[...],
                preferred_element_type=jnp.float32).astype(jnp.bfloat16)

        def rs_rdma(r, s):
            nbr = right if r == 0 else left
            send_c = (my - s) % N_DEV if r == 0 else (my + s) % N_DEV
            rdma = pltpu.make_async_remote_copy(
                src_ref=pbuf.at[rows(send_c), cols(r)],
                dst_ref=rs_buf.at[r, s],
                send_sem=send_sems.at[r, s],
                recv_sem=recv_sems.at[r, s],
                device_id=(nbr,),
                device_id_type=pl.DeviceIdType.MESH,
            )
            rdma.start()
            return rdma

        cp = load(my, 0)
        cp.wait()
        cp = load((my - 1) % N_DEV, 1)
        compute(my, 0)
        rs0 = [rs_rdma(0, 0), rs_rdma(1, 0)]
        cp.wait()
        cp = load((my + 1) % N_DEV, 0)
        compute((my - 1) % N_DEV, 1)
        cp.wait()
        cp = load((my + 2) % N_DEV, 1)
        compute((my + 1) % N_DEV, 0)
        cp.wait()
        compute((my + 2) % N_DEV, 1)

        pending = rs0
        for s in range(N_RS):
            for r, rdma in enumerate(pending):
                rdma.wait()
                recv_c = ((my - 1 - s) if r == 0 else (my + 1 + s)) % N_DEV
                pbuf[rows(recv_c), cols(r)] += rs_buf[r, s]
            if s + 1 < N_RS:
                pending = [rs_rdma(0, s + 1), rs_rdma(1, s + 1)]

        scale = (sx_ref[0] * sw_ref[0]).astype(jnp.bfloat16)
        own = ((my + 1) % N_DEV, (my - 1) % N_DEV)
        for r in range(2):
            v = pbuf[rows(own[r]), cols(r)]
            pbuf[rows(own[r]), cols(r)] = jnp.maximum(v * scale, 0)

        store_cps = []

        def store_half(c, r):
            k = len(store_cps)
            slot = k % 2
            if k >= 2:
                store_cps[k - 2].wait()
            stage[slot] = pbuf[rows(c), cols(r)].astype(jnp.float32)
            cp = pltpu.make_async_copy(
                stage.at[slot], out_hbm.at[rows(c), cols(r)],
                store_sems.at[slot])
            cp.start()
            store_cps.append(cp)

        def ag_rdma(r, t):
            nbr = right if r == 0 else left
            send_c = ((my + 1 - t) if r == 0 else (my - 1 + t)) % N_DEV
            rdma = pltpu.make_async_remote_copy(
                src_ref=pbuf.at[rows(send_c), cols(r)],
                dst_ref=pbuf.at[rows(send_c), cols(r)],
                send_sem=send_sems.at[r, N_RS + t],
                recv_sem=recv_sems.at[r, N_RS + t],
                device_id=(nbr,),
                device_id_type=pl.DeviceIdType.MESH,
            )
            rdma.start()
            return rdma

        pending = [ag_rdma(0, 0), ag_rdma(1, 0)]
        store_half(own[0], 0)
        store_half(own[1], 1)
        for t in range(N_RS):
            nxt = None
            for r, rdma in enumerate(pending):
                rdma.wait()
            if t + 1 < N_RS:
                nxt = [ag_rdma(0, t + 1), ag_rdma(1, t + 1)]
            for r in range(2):
                recv_c = ((my - t) if r == 0 else (my + t)) % N_DEV
                store_half(recv_c, r)
            pending = nxt

        store_cps[-2].wait()
        store_cps[-1].wait()

    return pl.pallas_call(
        body,
        out_shape=jax.ShapeDtypeStruct((m, n), jnp.float32),
        in_specs=[
            pl.BlockSpec(memory_space=pl.ANY),
            pl.BlockSpec(memory_space=pltpu.VMEM),
            pl.BlockSpec(memory_space=pltpu.SMEM),
            pl.BlockSpec(memory_space=pltpu.SMEM),
        ],
        out_specs=pl.BlockSpec(memory_space=pl.ANY),
        scratch_shapes=[
            pltpu.VMEM((2, m_per, k_per), jnp.float32),
            pltpu.VMEM((m, n), jnp.bfloat16),
            pltpu.VMEM((2, N_RS, m_per, n_half), jnp.bfloat16),
            pltpu.VMEM((2, m_per, n_half), jnp.float32),
            pltpu.SemaphoreType.DMA((2,)),
            pltpu.SemaphoreType.DMA((2,)),
            pltpu.SemaphoreType.DMA((2, 2 * N_RS)),
            pltpu.SemaphoreType.DMA((2, 2 * N_RS)),
        ],
        compiler_params=pltpu.CompilerParams(
            collective_id=0, vmem_limit_bytes=64 * 1024 * 1024),
    )(x, w_mat, scale_x, scale_w)


# device time: 177583 ns/iter; 1.9902x vs baseline; 1.0513x over previous
import jax
import jax.numpy as jnp
from jax import lax
from jax.experimental import pallas as pl
from jax.experimental.pallas import tpu as pltpu

N_DEV = 4
N_RS = N_DEV - 1
N_RINGS = 4


def kernel(x, w_mat, scale_x, scale_w):
    m, k_per = x.shape
    _, n = w_mat.shape
    m_per = m // N_DEV
    n_q = n // N_RINGS

    def body(x_hbm, w_ref, sx_ref, sw_ref, out_hbm,
             xchunk, pbuf, rs_buf, stage, load_sems, store_sems,
             send_sems, recv_sems):
        my = lax.axis_index("i")
        left = (my - 1) % N_DEV
        right = (my + 1) % N_DEV

        def rows(c):
            return pl.ds(c * m_per, m_per)

        def cols(q):
            return pl.ds(q * n_q, n_q)

        barrier_sem = pltpu.get_barrier_semaphore()
        for nbr in (left, right):
            pl.semaphore_signal(
                barrier_sem, inc=1,
                device_id=(nbr,), device_id_type=pl.DeviceIdType.MESH,
            )
        pl.semaphore_wait(barrier_sem, 2)

        def load(c, slot):
            cp = pltpu.make_async_copy(
                x_hbm.at[rows(c), :], xchunk.at[slot], load_sems.at[slot])
            cp.start()
            return cp

        def compute(c, slot):
            pbuf[rows(c), :] = jnp.dot(
                xchunk[slot], w_ref[...],
                preferred_element_type=jnp.float32).astype(jnp.bfloat16)

        def rs_rdma(q, s):
            rightward = q % 2 == 0
            nbr = right if rightward else left
            send_c = (my - s) % N_DEV if rightward else (my + s) % N_DEV
            rdma = pltpu.make_async_remote_copy(
                src_ref=pbuf.at[rows(send_c), cols(q)],
                dst_ref=rs_buf.at[q, s],
                send_sem=send_sems.at[q, s],
                recv_sem=recv_sems.at[q, s],
                device_id=(nbr,),
                device_id_type=pl.DeviceIdType.MESH,
            )
            rdma.start()
            return rdma

        cp = load(my, 0)
        cp.wait()
        cp = load((my - 1) % N_DEV, 1)
        compute(my, 0)
        pending = [rs_rdma(q, 0) for q in range(N_RINGS)]
        cp.wait()
        cp = load((my + 1) % N_DEV, 0)
        compute((my - 1) % N_DEV, 1)
        cp.wait()
        cp = load((my + 2) % N_DEV, 1)
        compute((my + 1) % N_DEV, 0)
        cp.wait()
        compute((my + 2) % N_DEV, 1)

        for s in range(N_RS):
            nxt = [None] * N_RINGS
            for q, rdma in enumerate(pending):
                rdma.wait()
                recv_c = ((my - 1 - s) if q % 2 == 0 else
                          (my + 1 + s)) % N_DEV
                pbuf[rows(recv_c), cols(q)] += rs_buf[q, s]
                if s + 1 < N_RS:
                    nxt[q] = rs_rdma(q, s + 1)
            pending = nxt

        scale = (sx_ref[0] * sw_ref[0]).astype(jnp.bfloat16)

        def own(q):
            return ((my + 1) if q % 2 == 0 else (my - 1)) % N_DEV

        for q in range(N_RINGS):
            v = pbuf[rows(own(q)), cols(q)]
            pbuf[rows(own(q)), cols(q)] = jnp.maximum(v * scale, 0)

        store_cps = []

        def store_quarter(c, q):
            k = len(store_cps)
            slot = k % 2
            if k >= 2:
                store_cps[k - 2].wait()
            stage[slot] = pbuf[rows(c), cols(q)].astype(jnp.float32)
            cp = pltpu.make_async_copy(
                stage.at[slot], out_hbm.at[rows(c), cols(q)],
                store_sems.at[slot])
            cp.start()
            store_cps.append(cp)

        def ag_rdma(q, t):
            rightward = q % 2 == 0
            nbr = right if rightward else left
            send_c = ((my + 1 - t) if rightward else (my - 1 + t)) % N_DEV
            rdma = pltpu.make_async_remote_copy(
                src_ref=pbuf.at[rows(send_c), cols(q)],
                dst_ref=pbuf.at[rows(send_c), cols(q)],
                send_sem=send_sems.at[q, N_RS + t],
                recv_sem=recv_sems.at[q, N_RS + t],
                device_id=(nbr,),
                device_id_type=pl.DeviceIdType.MESH,
            )
            rdma.start()
            return rdma

        pending = [ag_rdma(q, 0) for q in range(N_RINGS)]
        for q in range(N_RINGS):
            store_quarter(own(q), q)
        for t in range(N_RS):
            nxt = [None] * N_RINGS
            for q, rdma in enumerate(pending):
                rdma.wait()
                if t + 1 < N_RS:
                    nxt[q] = ag_rdma(q, t + 1)
                recv_c = ((my - t) if q % 2 == 0 else (my + t)) % N_DEV
                store_quarter(recv_c, q)
            pending = nxt

        store_cps[-2].wait()
        store_cps[-1].wait()

    return pl.pallas_call(
        body,
        out_shape=jax.ShapeDtypeStruct((m, n), jnp.float32),
        in_specs=[
            pl.BlockSpec(memory_space=pl.ANY),
            pl.BlockSpec(memory_space=pltpu.VMEM),
            pl.BlockSpec(memory_space=pltpu.SMEM),
            pl.BlockSpec(memory_space=pltpu.SMEM),
        ],
        out_specs=pl.BlockSpec(memory_space=pl.ANY),
        scratch_shapes=[
            pltpu.VMEM((2, m_per, k_per), jnp.float32),
            pltpu.VMEM((m, n), jnp.bfloat16),
            pltpu.VMEM((N_RINGS, N_RS, m_per, n_q), jnp.bfloat16),
            pltpu.VMEM((2, m_per, n_q), jnp.float32),
            pltpu.SemaphoreType.DMA((2,)),
            pltpu.SemaphoreType.DMA((2,)),
            pltpu.SemaphoreType.DMA((N_RINGS, 2 * N_RS)),
            pltpu.SemaphoreType.DMA((N_RINGS, 2 * N_RS)),
        ],
        compiler_params=pltpu.CompilerParams(
            collective_id=0, vmem_limit_bytes=64 * 1024 * 1024),
    )(x, w_mat, scale_x, scale_w)
